# baseline (speedup 1.0000x reference)
import jax
import jax.numpy as jnp
from jax import lax
from jax.experimental import pallas as pl
from jax.experimental.pallas import tpu as pltpu
from jax.experimental.pallas import tpu_sc as plsc

N = 10000
E = 320000
CIN = 128
HID = 256
COUT = 128
HEADS = 4

NC = 2
NS = 16
NW = NC * NS
LANES = 128
K = 80
TOTCH = K * NW
K0 = 56
K1 = K * 2 - K0
CH0 = NS * K0
KMAX = K1
EPAD = TOTCH * LANES
NPAD = 10240
RPT = NPAD // NS
BLK = 512
GRID = NPAD // BLK


def _deg_body(dst_hbm, ones_hbm, zeros_hbm, out_hbm, didx_v, ones_v, acc):
    c = lax.axis_index("c")
    s = lax.axis_index("s")
    base = s * RPT
    bc = lax.select(c == 0, s * K0, CH0 + s * K1)
    pltpu.sync_copy(zeros_hbm.at[pl.ds(base, RPT)], acc.at[pl.ds(base, RPT)])
    pltpu.sync_copy(dst_hbm.at[pl.ds(bc, KMAX)], didx_v)
    pltpu.sync_copy(ones_hbm, ones_v)
    plsc.subcore_barrier()

    def fire(k, carry):
        pltpu.sync_copy(ones_v, acc.at[didx_v.at[k]], add=True)
        return carry

    @pl.when(c == 0)
    def _():
        lax.fori_loop(0, K0, fire, 0)

    @pl.when(c == 1)
    def _():
        lax.fori_loop(0, K1, fire, 0)

    plsc.subcore_barrier()
    pltpu.sync_copy(acc.at[pl.ds(base, RPT)], out_hbm.at[c, pl.ds(base, RPT)])


def _sc_degree(dst_idx, ones128, zeros128):
    mesh = plsc.VectorSubcoreMesh(core_axis_name="c", subcore_axis_name="s")
    f = pl.kernel(
        _deg_body,
        out_type=jax.ShapeDtypeStruct((NC, NPAD, 128), jnp.float32),
        mesh=mesh,
        scratch_types=[
            pltpu.VMEM((KMAX, LANES), jnp.int32),
            pltpu.VMEM((LANES, 128), jnp.float32),
            pltpu.VMEM_SHARED((NPAD, 128), jnp.float32),
        ],
    )
    return f(dst_idx, ones128, zeros128)


def _make_prop_body(num_tables):
    def body(*refs):
        tables = refs[:num_tables]
        src_hbm, dst_hbm, zeros_hbm, out_hbm = refs[num_tables:num_tables + 4]
        sidx_v, didx_v, buf, acc, sem = refs[num_tables + 4:]
        c = lax.axis_index("c")
        s = lax.axis_index("s")
        base = s * RPT
        bc = lax.select(c == 0, s * K0, CH0 + s * K1)
        pltpu.sync_copy(src_hbm.at[pl.ds(bc, KMAX)], sidx_v)
        pltpu.sync_copy(dst_hbm.at[pl.ds(bc, KMAX)], didx_v)
        for h in range(num_tables):
            table = tables[h]
            pltpu.sync_copy(zeros_hbm.at[pl.ds(base, RPT)],
                            acc.at[pl.ds(base, RPT)])
            plsc.subcore_barrier()

            def step(k, carry, table=table):
                pltpu.async_copy(table.at[sidx_v.at[k]], buf, sem).wait()
                pltpu.sync_copy(buf, acc.at[didx_v.at[k]], add=True)
                return carry

            @pl.when(c == 0)
            def _():
                lax.fori_loop(0, K0, step, 0)

            @pl.when(c == 1)
            def _():
                lax.fori_loop(0, K1, step, 0)

            plsc.subcore_barrier()
            pltpu.sync_copy(acc.at[pl.ds(base, RPT)],
                            out_hbm.at[c, h, pl.ds(base, RPT)])

    return body


def _sc_propagate(tables, src_idx, dst_idx, zeros128):
    num = len(tables)
    mesh = plsc.VectorSubcoreMesh(core_axis_name="c", subcore_axis_name="s")
    f = pl.kernel(
        _make_prop_body(num),
        out_type=jax.ShapeDtypeStruct((NC, num, NPAD, 128), jnp.float32),
        mesh=mesh,
        scratch_types=[
            pltpu.VMEM((KMAX, LANES), jnp.int32),
            pltpu.VMEM((KMAX, LANES), jnp.int32),
            pltpu.VMEM((LANES, 128), jnp.float32),
            pltpu.VMEM_SHARED((NPAD, 128), jnp.float32),
            pltpu.SemaphoreType.DMA,
        ],
    )
    return f(*tables, src_idx, dst_idx, zeros128)


def _tc1_body(degp_ref, x_ref, xs_ref, dinvb_ref):
    deg = degp_ref[0, :, 0:1] + degp_ref[1, :, 0:1] + 1.0
    dinv = lax.rsqrt(deg)
    xs_ref[...] = x_ref[...] * dinv
    dinvb_ref[...] = jnp.broadcast_to(dinv, (BLK, 128))


def _tc_scale(degp, x_pad):
    return pl.pallas_call(
        _tc1_body,
        grid=(GRID,),
        in_specs=[
            pl.BlockSpec((NC, BLK, 128), lambda i: (0, i, 0)),
            pl.BlockSpec((BLK, 128), lambda i: (i, 0)),
        ],
        out_specs=[
            pl.BlockSpec((BLK, 128), lambda i: (i, 0)),
            pl.BlockSpec((BLK, 128), lambda i: (i, 0)),
        ],
        out_shape=[
            jax.ShapeDtypeStruct((NPAD, 128), jnp.float32),
            jax.ShapeDtypeStruct((NPAD, 128), jnp.float32),
        ],
    )(degp, x_pad)


def _tc2_body(p_ref, xs_ref, dinvb_ref, w1_ref, b1_ref, w2_ref, ys_ref):
    dinv = dinvb_ref[...]
    z = (p_ref[0] + p_ref[1] + xs_ref[...]) * dinv
    hh = lax.dot_general(z, w1_ref[0], (((1,), (0,)), ((), ())),
                         precision=lax.Precision.HIGHEST,
                         preferred_element_type=jnp.float32)
    hh = jnp.maximum(hh + b1_ref[0], 0.0)
    y = lax.dot_general(hh, w2_ref[0], (((1,), (0,)), ((), ())),
                        precision=lax.Precision.HIGHEST,
                        preferred_element_type=jnp.float32)
    ys_ref[0] = y * dinv


def _tc_heads(p1, xs, dinvb, W1, b1, W2):
    return pl.pallas_call(
        _tc2_body,
        grid=(HEADS, GRID),
        in_specs=[
            pl.BlockSpec((NC, BLK, 128), lambda h, i: (0, i, 0)),
            pl.BlockSpec((BLK, 128), lambda h, i: (i, 0)),
            pl.BlockSpec((BLK, 128), lambda h, i: (i, 0)),
            pl.BlockSpec((1, CIN, HID), lambda h, i: (h, 0, 0)),
            pl.BlockSpec((1, 1, HID), lambda h, i: (h, 0, 0)),
            pl.BlockSpec((1, HID, COUT), lambda h, i: (h, 0, 0)),
        ],
        out_specs=pl.BlockSpec((1, BLK, 128), lambda h, i: (h, i, 0)),
        out_shape=jax.ShapeDtypeStruct((HEADS, NPAD, 128), jnp.float32),
    )(p1, xs, dinvb, W1, b1, W2)


def _tc3_body(p2_ref, ys_ref, dinvb_ref, b2_ref, out_ref):
    o = (p2_ref[0, 0] + p2_ref[1, 0] + ys_ref[0]) * dinvb_ref[...]
    out_ref[0] = o + b2_ref[0]


def _tc_combine(p2, ys, dinvb, b2):
    return pl.pallas_call(
        _tc3_body,
        grid=(HEADS, GRID),
        in_specs=[
            pl.BlockSpec((NC, 1, BLK, 128), lambda h, i: (0, h, i, 0)),
            pl.BlockSpec((1, BLK, 128), lambda h, i: (h, i, 0)),
            pl.BlockSpec((BLK, 128), lambda h, i: (i, 0)),
            pl.BlockSpec((1, 1, 128), lambda h, i: (h, 0, 0)),
        ],
        out_specs=pl.BlockSpec((1, BLK, 128), lambda h, i: (h, i, 0)),
        out_shape=jax.ShapeDtypeStruct((HEADS, NPAD, 128), jnp.float32),
    )(p2, ys, dinvb, b2)


def kernel(x, edge_index, W1, b1, W2, b2):
    src = edge_index[0].astype(jnp.int32)
    dst = edge_index[1].astype(jnp.int32)
    pad = EPAD - E
    src_p = jnp.concatenate([src, jnp.zeros((pad,), jnp.int32)])
    dst_p = jnp.concatenate([dst, jnp.full((pad,), N, jnp.int32)])
    src_p = src_p.reshape(TOTCH, LANES)
    dst_p = dst_p.reshape(TOTCH, LANES)
    x_pad = jnp.pad(x, ((0, NPAD - N), (0, 0)))
    zeros128 = jnp.zeros((NPAD, 128), jnp.float32)
    ones128 = jnp.ones((LANES, 128), jnp.float32)

    degp = _sc_degree(dst_p, ones128, zeros128)
    xs, dinvb = _tc_scale(degp, x_pad)
    p1 = _sc_propagate([xs], src_p, dst_p, zeros128)
    ys = _tc_heads(p1[:, 0], xs, dinvb, W1,
                   b1.reshape(HEADS, 1, HID), W2)
    p2 = _sc_propagate([ys[h] for h in range(HEADS)],
                       src_p, dst_p, zeros128)
    out = _tc_combine(p2, ys, dinvb,
                      b2.reshape(HEADS, 1, COUT))
    return out[:, :N, :]


# speedup vs baseline: 1.0768x; 1.0768x over previous
import jax
import jax.numpy as jnp
from jax import lax
from jax.experimental import pallas as pl
from jax.experimental.pallas import tpu as pltpu
from jax.experimental.pallas import tpu_sc as plsc

N = 10000
E = 320000
CIN = 128
HID = 256
COUT = 128
HEADS = 4

NC = 2
NS = 16
NW = NC * NS
LANES = 128
K = 80
TOTCH = K * NW
K0 = 104
K1 = K * 2 - K0
CH0 = NS * K0
KMAX = max(K0, K1)
TOTA = TOTCH + (KMAX - min(K0, K1))
EPAD = TOTA * LANES
NPAD = 10240
RPT = NPAD // NS
BLK = 512
GRID = NPAD // BLK


def _deg_body(dst_hbm, ones_hbm, zeros_hbm, out_hbm, didx_v, ones_v, acc):
    c = lax.axis_index("c")
    s = lax.axis_index("s")
    base = s * RPT
    bc = lax.select(c == 0, s * K0, CH0 + s * K1)
    pltpu.sync_copy(zeros_hbm.at[pl.ds(base, RPT)], acc.at[pl.ds(base, RPT)])
    pltpu.sync_copy(dst_hbm.at[pl.ds(bc, KMAX)], didx_v)
    pltpu.sync_copy(ones_hbm, ones_v)
    plsc.subcore_barrier()

    def fire(k, carry):
        pltpu.sync_copy(ones_v, acc.at[didx_v.at[k]], add=True)
        return carry

    @pl.when(c == 0)
    def _():
        lax.fori_loop(0, K0, fire, 0)

    @pl.when(c == 1)
    def _():
        lax.fori_loop(0, K1, fire, 0)

    plsc.subcore_barrier()
    pltpu.sync_copy(acc.at[pl.ds(base, RPT)], out_hbm.at[c, pl.ds(base, RPT)])


def _sc_degree(dst_idx, ones128, zeros128):
    mesh = plsc.VectorSubcoreMesh(core_axis_name="c", subcore_axis_name="s")
    f = pl.kernel(
        _deg_body,
        out_type=jax.ShapeDtypeStruct((NC, NPAD, 128), jnp.float32),
        mesh=mesh,
        scratch_types=[
            pltpu.VMEM((KMAX, LANES), jnp.int32),
            pltpu.VMEM((LANES, 128), jnp.float32),
            pltpu.VMEM_SHARED((NPAD, 128), jnp.float32),
        ],
    )
    return f(dst_idx, ones128, zeros128)


def _make_prop_body(num_tables):
    def body(*refs):
        tables = refs[:num_tables]
        src_hbm, dst_hbm, zeros_hbm, out_hbm = refs[num_tables:num_tables + 4]
        sidx_v, didx_v, buf, acc, sem = refs[num_tables + 4:]
        c = lax.axis_index("c")
        s = lax.axis_index("s")
        base = s * RPT
        bc = lax.select(c == 0, s * K0, CH0 + s * K1)
        pltpu.sync_copy(src_hbm.at[pl.ds(bc, KMAX)], sidx_v)
        pltpu.sync_copy(dst_hbm.at[pl.ds(bc, KMAX)], didx_v)
        for h in range(num_tables):
            table = tables[h]
            pltpu.sync_copy(zeros_hbm.at[pl.ds(base, RPT)],
                            acc.at[pl.ds(base, RPT)])
            plsc.subcore_barrier()

            def step(k, carry, table=table):
                pltpu.async_copy(table.at[sidx_v.at[k]], buf, sem).wait()
                pltpu.sync_copy(buf, acc.at[didx_v.at[k]], add=True)
                return carry

            @pl.when(c == 0)
            def _():
                lax.fori_loop(0, K0, step, 0)

            @pl.when(c == 1)
            def _():
                lax.fori_loop(0, K1, step, 0)

            plsc.subcore_barrier()
            pltpu.sync_copy(acc.at[pl.ds(base, RPT)],
                            out_hbm.at[c, h, pl.ds(base, RPT)])

    return body


def _sc_propagate(tables, src_idx, dst_idx, zeros128):
    num = len(tables)
    mesh = plsc.VectorSubcoreMesh(core_axis_name="c", subcore_axis_name="s")
    f = pl.kernel(
        _make_prop_body(num),
        out_type=jax.ShapeDtypeStruct((NC, num, NPAD, 128), jnp.float32),
        mesh=mesh,
        scratch_types=[
            pltpu.VMEM((KMAX, LANES), jnp.int32),
            pltpu.VMEM((KMAX, LANES), jnp.int32),
            pltpu.VMEM((LANES, 128), jnp.float32),
            pltpu.VMEM_SHARED((NPAD, 128), jnp.float32),
            pltpu.SemaphoreType.DMA,
        ],
    )
    return f(*tables, src_idx, dst_idx, zeros128)


def _tc1_body(degp_ref, x_ref, xs_ref, dinvb_ref):
    deg = degp_ref[0, :, 0:1] + degp_ref[1, :, 0:1] + 1.0
    dinv = lax.rsqrt(deg)
    xs_ref[...] = x_ref[...] * dinv
    dinvb_ref[...] = jnp.broadcast_to(dinv, (BLK, 128))


def _tc_scale(degp, x_pad):
    return pl.pallas_call(
        _tc1_body,
        grid=(GRID,),
        in_specs=[
            pl.BlockSpec((NC, BLK, 128), lambda i: (0, i, 0)),
            pl.BlockSpec((BLK, 128), lambda i: (i, 0)),
        ],
        out_specs=[
            pl.BlockSpec((BLK, 128), lambda i: (i, 0)),
            pl.BlockSpec((BLK, 128), lambda i: (i, 0)),
        ],
        out_shape=[
            jax.ShapeDtypeStruct((NPAD, 128), jnp.float32),
            jax.ShapeDtypeStruct((NPAD, 128), jnp.float32),
        ],
    )(degp, x_pad)


def _tc2_body(p_ref, xs_ref, dinvb_ref, w1_ref, b1_ref, w2_ref, ys_ref):
    dinv = dinvb_ref[...]
    z = (p_ref[0] + p_ref[1] + xs_ref[...]) * dinv
    hh = lax.dot_general(z, w1_ref[0], (((1,), (0,)), ((), ())),
                         precision=lax.Precision.HIGHEST,
                         preferred_element_type=jnp.float32)
    hh = jnp.maximum(hh + b1_ref[0], 0.0)
    y = lax.dot_general(hh, w2_ref[0], (((1,), (0,)), ((), ())),
                        precision=lax.Precision.HIGHEST,
                        preferred_element_type=jnp.float32)
    ys_ref[0] = y * dinv


def _tc_heads(p1, xs, dinvb, W1, b1, W2):
    return pl.pallas_call(
        _tc2_body,
        grid=(HEADS, GRID),
        in_specs=[
            pl.BlockSpec((NC, BLK, 128), lambda h, i: (0, i, 0)),
            pl.BlockSpec((BLK, 128), lambda h, i: (i, 0)),
            pl.BlockSpec((BLK, 128), lambda h, i: (i, 0)),
            pl.BlockSpec((1, CIN, HID), lambda h, i: (h, 0, 0)),
            pl.BlockSpec((1, 1, HID), lambda h, i: (h, 0, 0)),
            pl.BlockSpec((1, HID, COUT), lambda h, i: (h, 0, 0)),
        ],
        out_specs=pl.BlockSpec((1, BLK, 128), lambda h, i: (h, i, 0)),
        out_shape=jax.ShapeDtypeStruct((HEADS, NPAD, 128), jnp.float32),
    )(p1, xs, dinvb, W1, b1, W2)


def _tc3_body(p2_ref, ys_ref, dinvb_ref, b2_ref, out_ref):
    o = (p2_ref[0, 0] + p2_ref[1, 0] + ys_ref[0]) * dinvb_ref[...]
    out_ref[0] = o + b2_ref[0]


def _tc_combine(p2, ys, dinvb, b2):
    return pl.pallas_call(
        _tc3_body,
        grid=(HEADS, GRID),
        in_specs=[
            pl.BlockSpec((NC, 1, BLK, 128), lambda h, i: (0, h, i, 0)),
            pl.BlockSpec((1, BLK, 128), lambda h, i: (h, i, 0)),
            pl.BlockSpec((BLK, 128), lambda h, i: (i, 0)),
            pl.BlockSpec((1, 1, 128), lambda h, i: (h, 0, 0)),
        ],
        out_specs=pl.BlockSpec((1, BLK, 128), lambda h, i: (h, i, 0)),
        out_shape=jax.ShapeDtypeStruct((HEADS, NPAD, 128), jnp.float32),
    )(p2, ys, dinvb, b2)


def kernel(x, edge_index, W1, b1, W2, b2):
    src = edge_index[0].astype(jnp.int32)
    dst = edge_index[1].astype(jnp.int32)
    pad = EPAD - E
    src_p = jnp.concatenate([src, jnp.zeros((pad,), jnp.int32)])
    dst_p = jnp.concatenate([dst, jnp.full((pad,), N, jnp.int32)])
    src_p = src_p.reshape(TOTA, LANES)
    dst_p = dst_p.reshape(TOTA, LANES)
    x_pad = jnp.pad(x, ((0, NPAD - N), (0, 0)))
    zeros128 = jnp.zeros((NPAD, 128), jnp.float32)
    ones128 = jnp.ones((LANES, 128), jnp.float32)

    degp = _sc_degree(dst_p, ones128, zeros128)
    xs, dinvb = _tc_scale(degp, x_pad)
    p1 = _sc_propagate([xs], src_p, dst_p, zeros128)
    ys = _tc_heads(p1[:, 0], xs, dinvb, W1,
                   b1.reshape(HEADS, 1, HID), W2)
    p2 = _sc_propagate([ys[h] for h in range(HEADS)],
                       src_p, dst_p, zeros128)
    out = _tc_combine(p2, ys, dinvb,
                      b2.reshape(HEADS, 1, COUT))
    return out[:, :N, :]


# speedup vs baseline: 1.4690x; 1.3643x over previous
import jax
import jax.numpy as jnp
from jax import lax
from jax.experimental import pallas as pl
from jax.experimental.pallas import tpu as pltpu
from jax.experimental.pallas import tpu_sc as plsc

N = 10000
E = 320000
CIN = 128
HID = 256
COUT = 128
HEADS = 4

NC = 2
NS = 16
NW = NC * NS
LANES = 128
K = 79
EPT = K * LANES
EPAD = EPT * NW
NPAD = 10240
RPT = NPAD // NS
BLK = 512
GRID = NPAD // BLK


def _deg_body(dst_hbm, ones_hbm, zeros_hbm, out_hbm, didx_v, ones_v, acc):
    c = lax.axis_index("c")
    s = lax.axis_index("s")
    wid = s * NC + c
    base = s * RPT
    pltpu.sync_copy(zeros_hbm.at[pl.ds(base, RPT)], acc.at[pl.ds(base, RPT)])
    pltpu.sync_copy(dst_hbm.at[wid], didx_v)
    pltpu.sync_copy(ones_hbm, ones_v)
    plsc.subcore_barrier()

    def fire(k, carry):
        pltpu.sync_copy(ones_v, acc.at[didx_v.at[k]], add=True)
        return carry

    lax.fori_loop(0, K, fire, 0)
    plsc.subcore_barrier()
    pltpu.sync_copy(acc.at[pl.ds(base, RPT)], out_hbm.at[c, pl.ds(base, RPT)])


def _sc_degree(dst_idx, ones128, zeros128):
    mesh = plsc.VectorSubcoreMesh(core_axis_name="c", subcore_axis_name="s")
    f = pl.kernel(
        _deg_body,
        out_type=jax.ShapeDtypeStruct((NC, NPAD, 128), jnp.float32),
        mesh=mesh,
        scratch_types=[
            pltpu.VMEM((K, LANES), jnp.int32),
            pltpu.VMEM((LANES, 128), jnp.float32),
            pltpu.VMEM_SHARED((NPAD, 128), jnp.float32),
        ],
    )
    return f(dst_idx, ones128, zeros128)


def _make_prop_body(num_tables):
    def body(*refs):
        tables = refs[:num_tables]
        src_hbm, dst_hbm, zeros_hbm, out_hbm = refs[num_tables:num_tables + 4]
        sidx_v, didx_v, buf, acc, sem = refs[num_tables + 4:]
        c = lax.axis_index("c")
        s = lax.axis_index("s")
        wid = s * NC + c
        base = s * RPT
        pltpu.sync_copy(src_hbm.at[wid], sidx_v)
        pltpu.sync_copy(dst_hbm.at[wid], didx_v)
        for h in range(num_tables):
            table = tables[h]
            pltpu.sync_copy(zeros_hbm.at[pl.ds(base, RPT)],
                            acc.at[pl.ds(base, RPT)])
            plsc.subcore_barrier()

            def step(k, carry, table=table):
                pltpu.async_copy(table.at[sidx_v.at[k]], buf, sem).wait()
                pltpu.sync_copy(buf, acc.at[didx_v.at[k]], add=True)
                return carry

            lax.fori_loop(0, K, step, 0)
            plsc.subcore_barrier()
            pltpu.sync_copy(acc.at[pl.ds(base, RPT)],
                            out_hbm.at[c, h, pl.ds(base, RPT)])

    return body


def _sc_propagate(tables, src_idx, dst_idx, zeros128):
    num = len(tables)
    mesh = plsc.VectorSubcoreMesh(core_axis_name="c", subcore_axis_name="s")
    f = pl.kernel(
        _make_prop_body(num),
        out_type=jax.ShapeDtypeStruct((NC, num, NPAD, 128), jnp.float32),
        mesh=mesh,
        scratch_types=[
            pltpu.VMEM((K, LANES), jnp.int32),
            pltpu.VMEM((K, LANES), jnp.int32),
            pltpu.VMEM((LANES, 128), jnp.float32),
            pltpu.VMEM_SHARED((NPAD, 128), jnp.float32),
            pltpu.SemaphoreType.DMA,
        ],
    )
    return f(*tables, src_idx, dst_idx, zeros128)


def _tc1_body(degp_ref, x_ref, xs_ref, dinvb_ref):
    deg = degp_ref[0, :, 0:1] + degp_ref[1, :, 0:1] + 1.0
    dinv = lax.rsqrt(deg)
    xs_ref[...] = x_ref[...] * dinv
    dinvb_ref[...] = jnp.broadcast_to(dinv, (BLK, 128))


def _tc_scale(degp, x_pad):
    return pl.pallas_call(
        _tc1_body,
        grid=(GRID,),
        in_specs=[
            pl.BlockSpec((NC, BLK, 128), lambda i: (0, i, 0)),
            pl.BlockSpec((BLK, 128), lambda i: (i, 0)),
        ],
        out_specs=[
            pl.BlockSpec((BLK, 128), lambda i: (i, 0)),
            pl.BlockSpec((BLK, 128), lambda i: (i, 0)),
        ],
        out_shape=[
            jax.ShapeDtypeStruct((NPAD, 128), jnp.float32),
            jax.ShapeDtypeStruct((NPAD, 128), jnp.float32),
        ],
    )(degp, x_pad)


def _tc2_body(p_ref, xs_ref, dinvb_ref, w1_ref, b1_ref, w2_ref, ys_ref):
    dinv = dinvb_ref[...]
    z = (p_ref[0] + p_ref[1] + xs_ref[...]) * dinv
    hh = lax.dot_general(z, w1_ref[0], (((1,), (0,)), ((), ())),
                         precision=lax.Precision.HIGHEST,
                         preferred_element_type=jnp.float32)
    hh = jnp.maximum(hh + b1_ref[0], 0.0)
    y = lax.dot_general(hh, w2_ref[0], (((1,), (0,)), ((), ())),
                        precision=lax.Precision.HIGHEST,
                        preferred_element_type=jnp.float32)
    ys_ref[0] = y * dinv


def _tc_heads(p1, xs, dinvb, W1, b1, W2):
    return pl.pallas_call(
        _tc2_body,
        grid=(HEADS, GRID),
        in_specs=[
            pl.BlockSpec((NC, BLK, 128), lambda h, i: (0, i, 0)),
            pl.BlockSpec((BLK, 128), lambda h, i: (i, 0)),
            pl.BlockSpec((BLK, 128), lambda h, i: (i, 0)),
            pl.BlockSpec((1, CIN, HID), lambda h, i: (h, 0, 0)),
            pl.BlockSpec((1, 1, HID), lambda h, i: (h, 0, 0)),
            pl.BlockSpec((1, HID, COUT), lambda h, i: (h, 0, 0)),
        ],
        out_specs=pl.BlockSpec((1, BLK, 128), lambda h, i: (h, i, 0)),
        out_shape=jax.ShapeDtypeStruct((HEADS, NPAD, 128), jnp.float32),
    )(p1, xs, dinvb, W1, b1, W2)


def _tc3_body(p2_ref, ys_ref, dinvb_ref, b2_ref, out_ref):
    o = (p2_ref[0, 0] + p2_ref[1, 0] + ys_ref[0]) * dinvb_ref[...]
    out_ref[0] = o + b2_ref[0]


def _tc_combine(p2, ys, dinvb, b2):
    return pl.pallas_call(
        _tc3_body,
        grid=(HEADS, GRID),
        in_specs=[
            pl.BlockSpec((NC, 1, BLK, 128), lambda h, i: (0, h, i, 0)),
            pl.BlockSpec((1, BLK, 128), lambda h, i: (h, i, 0)),
            pl.BlockSpec((BLK, 128), lambda h, i: (i, 0)),
            pl.BlockSpec((1, 1, 128), lambda h, i: (h, 0, 0)),
        ],
        out_specs=pl.BlockSpec((1, BLK, 128), lambda h, i: (h, i, 0)),
        out_shape=jax.ShapeDtypeStruct((HEADS, NPAD, 128), jnp.float32),
    )(p2, ys, dinvb, b2)


def kernel(x, edge_index, W1, b1, W2, b2):
    src = edge_index[0].astype(jnp.int32)
    dst = edge_index[1].astype(jnp.int32)
    pad = EPAD - E
    src_p = jnp.concatenate([src, jnp.zeros((pad,), jnp.int32)])
    dst_p = jnp.concatenate([dst, jnp.full((pad,), N, jnp.int32)])
    src_p = src_p.reshape(NW, K, LANES)
    dst_p = dst_p.reshape(NW, K, LANES)
    x_pad = jnp.pad(x, ((0, NPAD - N), (0, 0)))
    zeros128 = jnp.zeros((NPAD, 128), jnp.float32)
    ones128 = jnp.ones((LANES, 128), jnp.float32)

    degp = _sc_degree(dst_p, ones128, zeros128)
    xs, dinvb = _tc_scale(degp, x_pad)
    p1 = _sc_propagate([xs], src_p, dst_p, zeros128)
    ys = _tc_heads(p1[:, 0], xs, dinvb, W1,
                   b1.reshape(HEADS, 1, HID), W2)
    p2 = _sc_propagate([ys[h] for h in range(HEADS)],
                       src_p, dst_p, zeros128)
    out = _tc_combine(p2, ys, dinvb,
                      b2.reshape(HEADS, 1, COUT))
    return out[:, :N, :]
